# Initial kernel scaffold; baseline (speedup 1.0000x reference)
import jax
import jax.numpy as jnp
from jax.experimental import pallas as pl


def kernel(x, edge_index, batch, W_l0, W_r0, b0, W_l1, W_r1, b1, W_l2, W_r2, b2, W_ro, b_ro):
    raise NotImplementedError("write your pallas kernel here")


# speedup vs baseline: 33.2773x; 33.2773x over previous
import functools

import jax
import jax.numpy as jnp
from jax import lax
from jax.experimental import pallas as pl
from jax.experimental.pallas import tpu as pltpu
from jax.experimental.pallas import tpu_sc as plsc

N = 10000
E = 320000
D = 128
L = 16
G = 64

NC = 2
NS = 16
CW = 125
CH = 80
KG = 16
NGROUPS = CH // KG
NP = 10240
PR = NP * L // 128
RPS = NP // NS

_MESH = plsc.VectorSubcoreMesh(
    core_axis_name="c", subcore_axis_name="s", num_cores=NC, num_subcores=NS)

_SC_PARAMS = pltpu.CompilerParams(use_tc_tiling_on_sc=False)


def _sc_agg_body(with_deg, p_hbm, edl_hbm, parts_hbm, dparts_hbm,
                 src_v, dst_v, rows_v, ones_v, zbuf, acc, dacc, gsem0, gsem1,
                 ssem0, ssem1):
    gsem = (gsem0, gsem1)
    ssem = (ssem0, ssem1)
    c = lax.axis_index("c")
    s = lax.axis_index("s")

    def zero_body(i, carry):
        zbuf[i, :] = jnp.zeros((L,), jnp.float32)
        return carry
    lax.fori_loop(0, RPS, zero_body, None)
    pltpu.sync_copy(zbuf, acc.at[pl.ds(s * RPS, RPS)])
    if with_deg:
        pltpu.sync_copy(zbuf, dacc.at[pl.ds(s * RPS, RPS)])

        def ones_body(i, carry):
            ones_v[i, :] = jnp.ones((L,), jnp.float32)
            return carry
        lax.fori_loop(0, CW, ones_body, None)

    pltpu.sync_copy(edl_hbm.at[0, c, s], src_v)
    pltpu.sync_copy(edl_hbm.at[1, c, s], dst_v)
    plsc.subcore_barrier()

    def fire_gathers(g):
        base, buf = g * KG, g % 2
        return [
            pltpu.async_copy(p_hbm.at[src_v.at[base + j]],
                             rows_v.at[buf, j], gsem[buf])
            for j in range(KG)
        ]

    def fire_scatters(g):
        base, buf = g * KG, g % 2
        puts = [
            pltpu.async_copy(rows_v.at[buf, j], acc.at[dst_v.at[base + j]],
                             ssem[buf], add=True)
            for j in range(KG)
        ]
        if with_deg:
            puts += [
                pltpu.async_copy(ones_v, dacc.at[dst_v.at[base + j]],
                                 ssem[buf], add=True)
                for j in range(KG)
            ]
        return puts

    gd = {0: fire_gathers(0)}
    sd = {}
    for g in range(1, NGROUPS):
        if g >= 2:
            for d in sd[g - 2]:
                d.wait()
        gd[g] = fire_gathers(g)
        for d in gd[g - 1]:
            d.wait()
        sd[g - 1] = fire_scatters(g - 1)
    for d in gd[NGROUPS - 1]:
        d.wait()
    sd[NGROUPS - 1] = fire_scatters(NGROUPS - 1)
    for d in sd[NGROUPS - 2]:
        d.wait()
    for d in sd[NGROUPS - 1]:
        d.wait()
    plsc.subcore_barrier()

    pltpu.sync_copy(acc.at[pl.ds(s * RPS, RPS)],
                    parts_hbm.at[c, pl.ds(s * RPS, RPS)])
    if with_deg:
        pltpu.sync_copy(dacc.at[pl.ds(s * RPS, RPS)],
                        dparts_hbm.at[c, pl.ds(s * RPS, RPS)])


@functools.partial(
    pl.kernel,
    out_type=(jax.ShapeDtypeStruct((NC, NP, L), jnp.float32),
              jax.ShapeDtypeStruct((NC, NP, L), jnp.float32)),
    mesh=_MESH,
    compiler_params=_SC_PARAMS,
    scratch_types=[
        pltpu.VMEM((CH, CW), jnp.int32),
        pltpu.VMEM((CH, CW), jnp.int32),
        pltpu.VMEM((2, KG, CW, L), jnp.float32),
        pltpu.VMEM((CW, L), jnp.float32),
        pltpu.VMEM((RPS, L), jnp.float32),
        pltpu.VMEM_SHARED((NP, L), jnp.float32),
        pltpu.VMEM_SHARED((NP, L), jnp.float32),
        pltpu.SemaphoreType.DMA,
        pltpu.SemaphoreType.DMA,
        pltpu.SemaphoreType.DMA,
        pltpu.SemaphoreType.DMA,
    ],
)
def _sc_agg_deg(p_hbm, edl_hbm, parts_hbm, dparts_hbm,
                src_v, dst_v, rows_v, ones_v, zbuf, acc, dacc,
                gsem0, gsem1, ssem0, ssem1):
    _sc_agg_body(True, p_hbm, edl_hbm, parts_hbm, dparts_hbm,
                 src_v, dst_v, rows_v, ones_v, zbuf, acc, dacc,
                 gsem0, gsem1, ssem0, ssem1)


@functools.partial(
    pl.kernel,
    out_type=jax.ShapeDtypeStruct((NC, NP, L), jnp.float32),
    mesh=_MESH,
    compiler_params=_SC_PARAMS,
    scratch_types=[
        pltpu.VMEM((CH, CW), jnp.int32),
        pltpu.VMEM((CH, CW), jnp.int32),
        pltpu.VMEM((2, KG, CW, L), jnp.float32),
        pltpu.VMEM((RPS, L), jnp.float32),
        pltpu.VMEM_SHARED((NP, L), jnp.float32),
        pltpu.SemaphoreType.DMA,
        pltpu.SemaphoreType.DMA,
        pltpu.SemaphoreType.DMA,
        pltpu.SemaphoreType.DMA,
    ],
)
def _sc_agg(p_hbm, edl_hbm, parts_hbm,
            src_v, dst_v, rows_v, zbuf, acc, gsem0, gsem1, ssem0, ssem1):
    _sc_agg_body(False, p_hbm, edl_hbm, parts_hbm, None,
                 src_v, dst_v, rows_v, None, zbuf, acc, None,
                 gsem0, gsem1, ssem0, ssem1)


def _tc_proj_body(x_ref, wl_ref, wr_ref, p_ref, q_ref):
    x = x_ref[...]
    p_ref[...] = jnp.dot(x, wl_ref[...], preferred_element_type=jnp.float32)
    q_ref[...] = jnp.dot(x, wr_ref[...], preferred_element_type=jnp.float32)


def _tc_proj(xr, wl8, wr8):
    return pl.pallas_call(
        _tc_proj_body,
        out_shape=(jax.ShapeDtypeStruct((PR, 128), jnp.float32),
                   jax.ShapeDtypeStruct((PR, 128), jnp.float32)),
    )(xr, wl8, wr8)


def _tc_combine_body(parts_ref, dparts_ref, q_ref, b_ref, wl_ref, wr_ref,
                     p_out, q_out):
    agg = parts_ref[0] + parts_ref[1]
    deg = dparts_ref[0] + dparts_ref[1]
    inv = 1.0 / jnp.maximum(deg, 1.0)
    h = jnp.maximum(agg * inv + q_ref[...] + b_ref[...], 0.0)
    p_out[...] = jnp.dot(h, wl_ref[...], preferred_element_type=jnp.float32)
    q_out[...] = jnp.dot(h, wr_ref[...], preferred_element_type=jnp.float32)


def _tc_combine(parts, dparts, q, b8, wl8, wr8):
    return pl.pallas_call(
        _tc_combine_body,
        out_shape=(jax.ShapeDtypeStruct((PR, 128), jnp.float32),
                   jax.ShapeDtypeStruct((PR, 128), jnp.float32)),
    )(parts, dparts, q, b8, wl8, wr8)


def _tc_final_body(parts_ref, dparts_ref, q_ref, b_ref, batb_ref, w128_ref,
                   sel_ref, bro_ref, out_ref):
    agg = parts_ref[0] + parts_ref[1]
    deg = dparts_ref[0] + dparts_ref[1]
    inv = 1.0 / jnp.maximum(deg, 1.0)
    h = jnp.maximum(agg * inv + q_ref[...] + b_ref[...], 0.0)
    y8 = jnp.dot(h * w128_ref[...], sel_ref[...],
                 preferred_element_type=jnp.float32)
    gids = lax.broadcasted_iota(jnp.int32, (G, PR), 0)
    pooled = jnp.zeros((G, 1), jnp.float32)
    for s in range(8):
        mask = (batb_ref[s:s + 1, :] == gids).astype(jnp.float32)
        pooled = pooled + jnp.dot(mask, y8[:, s:s + 1],
                                  preferred_element_type=jnp.float32)
    out_ref[...] = jax.nn.sigmoid(pooled + bro_ref[...])


def _tc_final(parts, dparts, q, b8, batb, w128, sel, bro):
    return pl.pallas_call(
        _tc_final_body,
        out_shape=jax.ShapeDtypeStruct((G, 1), jnp.float32),
    )(parts, dparts, q, b8, batb, w128, sel, bro)


def _kron8(w):
    return jnp.kron(jnp.eye(8, dtype=w.dtype), w)


def _tile8(b):
    return jnp.tile(b, 8).reshape(1, 128)


def kernel(x, edge_index, batch, W_l0, W_r0, b0, W_l1, W_r1, b1,
           W_l2, W_r2, b2, W_ro, b_ro):
    edl = edge_index.reshape(2, NC, NS, CH, CW)
    xr = jnp.pad(x.reshape(N // 8, 8 * D), ((0, PR - N // 8), (0, 0)))

    p0, q0 = _tc_proj(xr, _kron8(W_l0), _kron8(W_r0))
    parts0, dparts = _sc_agg_deg(p0.reshape(NP, L), edl)
    pv0 = parts0.reshape(NC, PR, 128)
    dv = dparts.reshape(NC, PR, 128)
    p1, q1 = _tc_combine(pv0, dv, q0, _tile8(b0), _kron8(W_l1), _kron8(W_r1))
    parts1 = _sc_agg(p1.reshape(NP, L), edl)
    p2, q2 = _tc_combine(parts1.reshape(NC, PR, 128), dv, q1, _tile8(b1),
                         _kron8(W_l2), _kron8(W_r2))
    parts2 = _sc_agg(p2.reshape(NP, L), edl)

    batb = jnp.pad(batch, (0, NP - N), constant_values=G).reshape(PR, 8).T
    sel = _kron8(jnp.ones((L, 1), jnp.float32))
    w128 = _tile8(W_ro[:, 0])
    return _tc_final(parts2.reshape(NC, PR, 128), dv, q2, _tile8(b2),
                     batb, w128, sel, b_ro.reshape(1, 1))


# speedup vs baseline: 35.4787x; 1.0662x over previous
import functools

import numpy as np

import jax
import jax.numpy as jnp
from jax import lax
from jax.experimental import pallas as pl
from jax.experimental.pallas import tpu as pltpu
from jax.experimental.pallas import tpu_sc as plsc

N = 10000
E = 320000
D = 128
L = 16
G = 64

NC = 2
NS = 16
CW = 125
CH = 80
KG = 16
NGROUPS = CH // KG
NP = 10240
PR = NP * L // 128
RPS = NP // NS

_MESH = plsc.VectorSubcoreMesh(
    core_axis_name="c", subcore_axis_name="s", num_cores=NC, num_subcores=NS)

_SC_PARAMS = pltpu.CompilerParams(use_tc_tiling_on_sc=False)


def _sc_agg_body(with_deg, p_hbm, edl_hbm, parts_hbm, dparts_hbm,
                 src_v, dst_v, rows_v, ones_v, zbuf, acc, dacc, gsem0, gsem1,
                 ssem0, ssem1):
    gsem = (gsem0, gsem1)
    ssem = (ssem0, ssem1)
    c = lax.axis_index("c")
    s = lax.axis_index("s")

    def zero_body(i, carry):
        zbuf[i, :] = jnp.zeros((L,), jnp.float32)
        return carry
    lax.fori_loop(0, RPS, zero_body, None)
    pltpu.sync_copy(zbuf, acc.at[pl.ds(s * RPS, RPS)])
    if with_deg:
        pltpu.sync_copy(zbuf, dacc.at[pl.ds(s * RPS, RPS)])

        def ones_body(i, carry):
            ones_v[i, :] = jnp.ones((L,), jnp.float32)
            return carry
        lax.fori_loop(0, CW, ones_body, None)

    pltpu.sync_copy(edl_hbm.at[0, c, s], src_v)
    pltpu.sync_copy(edl_hbm.at[1, c, s], dst_v)
    plsc.subcore_barrier()

    def fire_gathers(g):
        base, buf = g * KG, g % 2
        return [
            pltpu.async_copy(p_hbm.at[src_v.at[base + j]],
                             rows_v.at[buf, j], gsem[buf])
            for j in range(KG)
        ]

    def fire_scatters(g):
        base, buf = g * KG, g % 2
        puts = [
            pltpu.async_copy(rows_v.at[buf, j], acc.at[dst_v.at[base + j]],
                             ssem[buf], add=True)
            for j in range(KG)
        ]
        if with_deg:
            puts += [
                pltpu.async_copy(ones_v, dacc.at[dst_v.at[base + j]],
                                 ssem[buf], add=True)
                for j in range(KG)
            ]
        return puts

    gd = {0: fire_gathers(0)}
    sd = {}
    for g in range(1, NGROUPS):
        if g >= 2:
            for d in sd[g - 2]:
                d.wait()
        gd[g] = fire_gathers(g)
        for d in gd[g - 1]:
            d.wait()
        sd[g - 1] = fire_scatters(g - 1)
    for d in gd[NGROUPS - 1]:
        d.wait()
    sd[NGROUPS - 1] = fire_scatters(NGROUPS - 1)
    for d in sd[NGROUPS - 2]:
        d.wait()
    for d in sd[NGROUPS - 1]:
        d.wait()
    plsc.subcore_barrier()

    pltpu.sync_copy(acc.at[pl.ds(s * RPS, RPS)],
                    parts_hbm.at[c, pl.ds(s * RPS, RPS)])
    if with_deg:
        pltpu.sync_copy(dacc.at[pl.ds(s * RPS, RPS)],
                        dparts_hbm.at[c, pl.ds(s * RPS, RPS)])


@functools.partial(
    pl.kernel,
    out_type=(jax.ShapeDtypeStruct((NC, NP, L), jnp.float32),
              jax.ShapeDtypeStruct((NC, NP, L), jnp.float32)),
    mesh=_MESH,
    compiler_params=_SC_PARAMS,
    scratch_types=[
        pltpu.VMEM((CH, CW), jnp.int32),
        pltpu.VMEM((CH, CW), jnp.int32),
        pltpu.VMEM((2, KG, CW, L), jnp.float32),
        pltpu.VMEM((CW, L), jnp.float32),
        pltpu.VMEM((RPS, L), jnp.float32),
        pltpu.VMEM_SHARED((NP, L), jnp.float32),
        pltpu.VMEM_SHARED((NP, L), jnp.float32),
        pltpu.SemaphoreType.DMA,
        pltpu.SemaphoreType.DMA,
        pltpu.SemaphoreType.DMA,
        pltpu.SemaphoreType.DMA,
    ],
)
def _sc_agg_deg(p_hbm, edl_hbm, parts_hbm, dparts_hbm,
                src_v, dst_v, rows_v, ones_v, zbuf, acc, dacc,
                gsem0, gsem1, ssem0, ssem1):
    _sc_agg_body(True, p_hbm, edl_hbm, parts_hbm, dparts_hbm,
                 src_v, dst_v, rows_v, ones_v, zbuf, acc, dacc,
                 gsem0, gsem1, ssem0, ssem1)


@functools.partial(
    pl.kernel,
    out_type=jax.ShapeDtypeStruct((NC, NP, L), jnp.float32),
    mesh=_MESH,
    compiler_params=_SC_PARAMS,
    scratch_types=[
        pltpu.VMEM((CH, CW), jnp.int32),
        pltpu.VMEM((CH, CW), jnp.int32),
        pltpu.VMEM((2, KG, CW, L), jnp.float32),
        pltpu.VMEM((RPS, L), jnp.float32),
        pltpu.VMEM_SHARED((NP, L), jnp.float32),
        pltpu.SemaphoreType.DMA,
        pltpu.SemaphoreType.DMA,
        pltpu.SemaphoreType.DMA,
        pltpu.SemaphoreType.DMA,
    ],
)
def _sc_agg(p_hbm, edl_hbm, parts_hbm,
            src_v, dst_v, rows_v, zbuf, acc, gsem0, gsem1, ssem0, ssem1):
    _sc_agg_body(False, p_hbm, edl_hbm, parts_hbm, None,
                 src_v, dst_v, rows_v, None, zbuf, acc, None,
                 gsem0, gsem1, ssem0, ssem1)


def _tc_proj_body(x_ref, wl_ref, wr_ref, p_ref, q_ref):
    x = x_ref[...]
    pad = ((0, NP - N), (0, 0))
    p = jnp.dot(x, wl_ref[...], preferred_element_type=jnp.float32)
    q = jnp.dot(x, wr_ref[...], preferred_element_type=jnp.float32)
    p_ref[...] = jnp.pad(p, pad)
    q_ref[...] = jnp.pad(q, pad)


def _tc_proj(x, wl, wr):
    return pl.pallas_call(
        _tc_proj_body,
        out_shape=(jax.ShapeDtypeStruct((NP, L), jnp.float32),
                   jax.ShapeDtypeStruct((NP, L), jnp.float32)),
    )(x, wl, wr)


def _tc_combine_body(parts_ref, dparts_ref, q_ref, b_ref, wl_ref, wr_ref,
                     p_out, q_out):
    agg = parts_ref[0] + parts_ref[1]
    deg = dparts_ref[0] + dparts_ref[1]
    inv = 1.0 / jnp.maximum(deg, 1.0)
    h = jnp.maximum(agg * inv + q_ref[...] + b_ref[...], 0.0)
    p_out[...] = jnp.dot(h, wl_ref[...], preferred_element_type=jnp.float32)
    q_out[...] = jnp.dot(h, wr_ref[...], preferred_element_type=jnp.float32)


def _tc_combine(parts, dparts, q, b8, wl8, wr8):
    return pl.pallas_call(
        _tc_combine_body,
        out_shape=(jax.ShapeDtypeStruct((PR, 128), jnp.float32),
                   jax.ShapeDtypeStruct((PR, 128), jnp.float32)),
    )(parts, dparts, q, b8, wl8, wr8)


def _tc_final_body(parts_ref, dparts_ref, q_ref, b_ref, batb_ref, w128_ref,
                   sel_ref, bro_ref, out_ref):
    agg = parts_ref[0] + parts_ref[1]
    deg = dparts_ref[0] + dparts_ref[1]
    inv = 1.0 / jnp.maximum(deg, 1.0)
    h = jnp.maximum(agg * inv + q_ref[...] + b_ref[...], 0.0)
    y8 = jnp.dot(h * w128_ref[...], sel_ref[...],
                 preferred_element_type=jnp.float32)
    gids = lax.broadcasted_iota(jnp.int32, (G, PR), 0)
    pooled = jnp.zeros((G, 1), jnp.float32)
    for s in range(8):
        mask = (batb_ref[s:s + 1, :] == gids).astype(jnp.float32)
        pooled = pooled + jnp.dot(mask, y8[:, s:s + 1],
                                  preferred_element_type=jnp.float32)
    out_ref[...] = jax.nn.sigmoid(pooled + bro_ref[...])


def _tc_final(parts, dparts, q, b8, batb, w128, sel, bro):
    return pl.pallas_call(
        _tc_final_body,
        out_shape=jax.ShapeDtypeStruct((G, 1), jnp.float32),
    )(parts, dparts, q, b8, batb, w128, sel, bro)


def _kron8(w):
    return jnp.kron(jnp.eye(8, dtype=w.dtype), w)


def _tile8(b):
    return jnp.tile(b, 8).reshape(1, 128)


_SEL = np.kron(np.eye(8, dtype=np.float32),
               np.ones((L, 1), np.float32))


def kernel(x, edge_index, batch, W_l0, W_r0, b0, W_l1, W_r1, b1,
           W_l2, W_r2, b2, W_ro, b_ro):
    edl = edge_index.reshape(2, NC, NS, CH, CW)

    p0, q0 = _tc_proj(x, W_l0, W_r0)
    parts0, dparts = _sc_agg_deg(p0, edl)
    pv0 = parts0.reshape(NC, PR, 128)
    dv = dparts.reshape(NC, PR, 128)
    q0pk = q0.reshape(PR, 128)
    p1, q1 = _tc_combine(pv0, dv, q0pk, _tile8(b0), _kron8(W_l1),
                         _kron8(W_r1))
    parts1 = _sc_agg(p1.reshape(NP, L), edl)
    p2, q2 = _tc_combine(parts1.reshape(NC, PR, 128), dv, q1, _tile8(b1),
                         _kron8(W_l2), _kron8(W_r2))
    parts2 = _sc_agg(p2.reshape(NP, L), edl)

    batb = jnp.pad(batch, (0, NP - N), constant_values=G).reshape(PR, 8).T
    w128 = _tile8(W_ro[:, 0])
    return _tc_final(parts2.reshape(NC, PR, 128), dv, q2, _tile8(b2),
                     batb, w128, jnp.asarray(_SEL), b_ro.reshape(1, 1))
